# Initial kernel scaffold; baseline (speedup 1.0000x reference)
import jax
import jax.numpy as jnp
from jax.experimental import pallas as pl


def kernel(uv, tex0, tex1):
    raise NotImplementedError("write your pallas kernel here")


# speedup vs baseline: 1.1604x; 1.1604x over previous
import functools

import jax
import jax.numpy as jnp
from jax import lax
from jax.experimental import pallas as pl
from jax.experimental.pallas import tpu as pltpu
from jax.experimental.pallas import tpu_sc as plsc

NC = 2
NS = 16
L = 16
NW = NC * NS

TEXW = 1024
CH = 16
B = 128
GROUPS = B // L


def _tex_pool_sc(n):
    per_w = n // NW
    iters = per_w // B
    mesh = plsc.VectorSubcoreMesh(
        core_axis_name="c", subcore_axis_name="s",
        num_cores=NC, num_subcores=NS)

    @functools.partial(
        pl.kernel,
        out_type=jax.ShapeDtypeStruct((n, 2 * CH), jnp.float32),
        mesh=mesh,
        compiler_params=pltpu.CompilerParams(use_tc_tiling_on_sc=False),
        scratch_types=[
            pltpu.VMEM((B,), jnp.float32),
            pltpu.VMEM((B,), jnp.float32),
            pltpu.VMEM((4, B), jnp.int32),
            pltpu.VMEM((4, B), jnp.float32),
            pltpu.VMEM((8, B, CH), jnp.float32),
            pltpu.VMEM((B, 2 * CH), jnp.float32),
            pltpu.SemaphoreType.DMA,
        ],
    )
    def k(uvT_hbm, t0_hbm, t1_hbm, out_hbm, u_v, v_v, idx_v, w_v, g_v, o_v,
          sem):
        wid = lax.axis_index("s") * NC + lax.axis_index("c")
        base_w = wid * per_w

        @pl.loop(0, iters)
        def _iter(it):
            base = base_w + it * B
            pltpu.sync_copy(uvT_hbm.at[0, pl.ds(base, B)], u_v)
            pltpu.sync_copy(uvT_hbm.at[1, pl.ds(base, B)], v_v)
            for g in range(GROUPS):
                sl = pl.ds(g * L, L)
                u = u_v[sl] * float(TEXW - 1)
                v = v_v[sl] * float(TEXW - 1)
                x0 = u.astype(jnp.int32)
                y0 = v.astype(jnp.int32)
                x0 = jnp.minimum(jnp.maximum(x0, 0), TEXW - 1)
                y0 = jnp.minimum(jnp.maximum(y0, 0), TEXW - 1)
                x1 = jnp.minimum(x0 + 1, TEXW - 1)
                y1 = jnp.minimum(y0 + 1, TEXW - 1)
                wx = u - x0.astype(jnp.float32)
                wy = v - y0.astype(jnp.float32)
                r0 = y0 << 10
                r1 = y1 << 10
                idx_v[0, sl] = r0 + x0
                idx_v[1, sl] = r0 + x1
                idx_v[2, sl] = r1 + x0
                idx_v[3, sl] = r1 + x1
                cx = 1.0 - wx
                cy = 1.0 - wy
                w_v[0, sl] = cx * cy
                w_v[1, sl] = wx * cy
                w_v[2, sl] = cx * wy
                w_v[3, sl] = wx * wy
            cps = []
            for c4 in range(4):
                cps.append(
                    pltpu.async_copy(t0_hbm.at[idx_v.at[c4]], g_v.at[c4],
                                     sem))
            for c4 in range(4):
                cps.append(
                    pltpu.async_copy(t1_hbm.at[idx_v.at[c4]], g_v.at[4 + c4],
                                     sem))
            for cp in cps:
                cp.wait()
            for g in range(GROUPS):
                sl = pl.ds(g * L, L)
                w00 = w_v[0, sl]
                w01 = w_v[1, sl]
                w10 = w_v[2, sl]
                w11 = w_v[3, sl]
                for j in range(L):
                    p = g * L + j
                    a00 = w00[j]
                    a01 = w01[j]
                    a10 = w10[j]
                    a11 = w11[j]
                    o_v[p, 0:CH] = (g_v[0, p, :] * a00 + g_v[1, p, :] * a01
                                    + g_v[2, p, :] * a10 + g_v[3, p, :] * a11)
                    o_v[p, CH:2 * CH] = (g_v[4, p, :] * a00
                                         + g_v[5, p, :] * a01
                                         + g_v[6, p, :] * a10
                                         + g_v[7, p, :] * a11)
            pltpu.sync_copy(o_v, out_hbm.at[pl.ds(base, B)])

    return k


def kernel(uv, tex0, tex1):
    n = uv.shape[0]
    uvT = uv.T
    t0 = tex0.reshape(TEXW * TEXW, CH)
    t1 = tex1.reshape(TEXW * TEXW, CH)
    return _tex_pool_sc(n)(uvT, t0, t1)
